# baseline (speedup 1.0000x reference)
import functools

import jax
import jax.numpy as jnp
from jax import lax
from jax.experimental import layout as jlayout
from jax.experimental import pallas as pl
from jax.experimental.pallas import tpu as pltpu
from jax.experimental.pallas import tpu_sc as plsc

NC = 2
NS = 16
NW = NC * NS
L = 16

EMB_DIM = 64
ROW = 2 * EMB_DIM
VOCAB_PAD = 1024
CHUNK = 256


def _table_body(w2s_hbm, emb_hbm, table_hbm, w2s_v, emb_v, row_v):
    wid = lax.axis_index("s") * NC + lax.axis_index("c")
    n = VOCAB_PAD // NW
    base = wid * n
    pltpu.sync_copy(w2s_hbm, w2s_v)
    pltpu.sync_copy(emb_hbm, emb_v)
    iota = jnp.arange(L, dtype=jnp.int32)
    zero = jnp.zeros((L,), jnp.float32)
    for k in range(n):
        r = jnp.minimum(base + k, w2s_v.shape[0] - 1)
        c = plsc.load_gather(w2s_v, [jnp.zeros((L,), jnp.int32) + r])
        for h in range(EMB_DIM // L):
            v = plsc.load_gather(emb_v, [c * EMB_DIM + h * L + iota])
            row_v[k, pl.ds(h * L, L)] = v
            row_v[k, pl.ds(EMB_DIM + h * L, L)] = zero
    pltpu.sync_copy(row_v, table_hbm.at[pl.ds(base, n)])


def _gather_body(batch, hist, inp_hbm, table_hbm, out_hbm, idx_v, rows_v, sem):
    wid = lax.axis_index("s") * NC + lax.axis_index("c")
    per_w = batch // NW
    b0w = wid * per_w

    def chunk_body(i, carry):
        b = b0w + i
        off = pl.multiple_of(b * hist, 8)
        pltpu.sync_copy(inp_hbm.at[pl.ds(off, hist)], idx_v)
        for o, n in ((0, 128), (128, hist - 128)):
            pltpu.async_copy(
                table_hbm.at[idx_v.at[pl.ds(o, n)]],
                rows_v.at[pl.ds(o, n)],
                sem,
            ).wait()
        pltpu.sync_copy(rows_v, out_hbm.at[b])
        return carry

    lax.fori_loop(0, per_w, chunk_body, 0)


def _impl(inp, w2s, emb):
    batch, hist = inp.shape
    total = batch * hist
    inp_flat = inp.astype(jnp.int32).reshape(total)

    mesh = plsc.VectorSubcoreMesh(core_axis_name="c", subcore_axis_name="s")
    params = pltpu.CompilerParams(needs_layout_passes=False)

    table = pl.kernel(
        _table_body,
        out_type=jax.ShapeDtypeStruct((VOCAB_PAD, ROW), jnp.float32),
        mesh=mesh,
        compiler_params=params,
        scratch_types=[
            pltpu.VMEM((w2s.shape[0],), jnp.int32),
            pltpu.VMEM((emb.size,), jnp.float32),
            pltpu.VMEM((VOCAB_PAD // NW, ROW), jnp.float32),
        ],
    )(w2s.astype(jnp.int32), emb.reshape(-1))

    out = pl.kernel(
        functools.partial(_gather_body, batch, hist),
        out_type=jax.ShapeDtypeStruct((batch, hist, ROW), jnp.float32),
        mesh=mesh,
        compiler_params=params,
        scratch_types=[
            pltpu.VMEM((hist,), jnp.int32),
            pltpu.VMEM((hist, ROW), jnp.float32),
            pltpu.SemaphoreType.DMA,
        ],
    )(inp_flat, table)

    return out[:, :, :EMB_DIM]


_jitted = {}


def kernel(input, word2syllable, embedding):
    dev = jax.devices()[0]
    fn = _jitted.get(dev)
    if fn is None:
        fmt = jlayout.Format(
            jlayout.Layout(major_to_minor=(0, 1, 2)),
            jax.sharding.SingleDeviceSharding(dev),
        )
        fn = jax.jit(_impl, out_shardings=fmt)
        _jitted[dev] = fn
    return fn(input, word2syllable, embedding)


# speedup vs baseline: 1.2726x; 1.2726x over previous
import functools

import jax
import jax.numpy as jnp
from jax import lax
from jax.experimental import layout as jlayout
from jax.experimental import pallas as pl
from jax.experimental.pallas import tpu as pltpu
from jax.experimental.pallas import tpu_sc as plsc

NC = 2
NS = 16
NW = NC * NS
L = 16

EMB_DIM = 64
ROW = 2 * EMB_DIM
VOCAB_PAD = 1024
CB = 2
SUPER = 8


def _table_body(w2s_hbm, emb_hbm, table_hbm, w2s_v, emb_v, row_v):
    wid = lax.axis_index("s") * NC + lax.axis_index("c")
    n = VOCAB_PAD // NW
    base = wid * n
    pltpu.sync_copy(w2s_hbm, w2s_v)
    pltpu.sync_copy(emb_hbm, emb_v)
    iota = jnp.arange(L, dtype=jnp.int32)
    zero = jnp.zeros((L,), jnp.float32)
    for k in range(n):
        r = jnp.minimum(base + k, w2s_v.shape[0] - 1)
        c = plsc.load_gather(w2s_v, [jnp.zeros((L,), jnp.int32) + r])
        for h in range(EMB_DIM // L):
            v = plsc.load_gather(emb_v, [c * EMB_DIM + h * L + iota])
            row_v[k, pl.ds(h * L, L)] = v
            row_v[k, pl.ds(EMB_DIM + h * L, L)] = zero
    pltpu.sync_copy(row_v, table_hbm.at[pl.ds(base, n)])


def _gather_body(batch, hist, inp_hbm, table_hbm, out_hbm,
                 idx0, idx1, rows0, rows1,
                 sem_i0, sem_i1, sem_g0, sem_g1, sem_o0, sem_o1):
    wid = lax.axis_index("s") * NC + lax.axis_index("c")
    per_w = batch // NW
    b0w = wid * per_w
    n_super = per_w // SUPER
    subs = SUPER // CB
    iters = per_w // CB
    idx = (idx0, idx1)
    rows = (rows0, rows1)
    sem_i = (sem_i0, sem_i1)
    sem_g = (sem_g0, sem_g1)
    sem_o = (sem_o0, sem_o1)
    splits = ((0, 128), (128, hist - 128))

    def idx_copy(k, buf):
        return pltpu.make_async_copy(
            inp_hbm.at[pl.ds(pl.multiple_of((b0w + k * SUPER), SUPER), SUPER)],
            idx[buf], sem_i[buf])

    def out_copy(j, buf):
        b0 = b0w + j * CB
        return pltpu.make_async_copy(
            rows[buf].reshape(CB, hist, ROW), out_hbm.at[pl.ds(b0, CB)],
            sem_o[buf])

    idx_copy(0, 0).start()

    def outer(ko, carry):
        for kk in range(2):
            k = 2 * ko + kk
            idx_copy(k, kk).wait()

            @pl.when(k + 1 < n_super)
            def _():
                idx_copy(k + 1, 1 - kk).start()

            for s in range(subs):
                j = k * subs + s
                sb = s % 2

                @pl.when(j >= 2)
                def _():
                    out_copy(j - 2, sb).wait()

                descs = []
                for r in range(CB):
                    rr = s * CB + r
                    for o, n in splits:
                        descs.append(pltpu.async_copy(
                            table_hbm.at[idx[kk].at[rr, pl.ds(o, n)]],
                            rows[sb].at[pl.ds(r * hist + o, n)],
                            sem_g[sb]))
                for d in descs:
                    d.wait()
                out_copy(j, sb).start()
        return carry

    lax.fori_loop(0, n_super // 2, outer, 0)
    out_copy(iters - 2, 0).wait()
    out_copy(iters - 1, 1).wait()


def _impl(inp, w2s, emb):
    batch, hist = inp.shape

    mesh = plsc.VectorSubcoreMesh(core_axis_name="c", subcore_axis_name="s")
    params = pltpu.CompilerParams(needs_layout_passes=False)

    table = pl.kernel(
        _table_body,
        out_type=jax.ShapeDtypeStruct((VOCAB_PAD, ROW), jnp.float32),
        mesh=mesh,
        compiler_params=params,
        scratch_types=[
            pltpu.VMEM((w2s.shape[0],), jnp.int32),
            pltpu.VMEM((emb.size,), jnp.float32),
            pltpu.VMEM((VOCAB_PAD // NW, ROW), jnp.float32),
        ],
    )(w2s.astype(jnp.int32), emb.reshape(-1))

    out = pl.kernel(
        functools.partial(_gather_body, batch, hist),
        out_type=jax.ShapeDtypeStruct((batch, hist, ROW), jnp.float32),
        mesh=mesh,
        compiler_params=params,
        scratch_types=[
            pltpu.VMEM((SUPER, hist), jnp.int32),
            pltpu.VMEM((SUPER, hist), jnp.int32),
            pltpu.VMEM((CB * hist, ROW), jnp.float32),
            pltpu.VMEM((CB * hist, ROW), jnp.float32),
            pltpu.SemaphoreType.DMA,
            pltpu.SemaphoreType.DMA,
            pltpu.SemaphoreType.DMA,
            pltpu.SemaphoreType.DMA,
            pltpu.SemaphoreType.DMA,
            pltpu.SemaphoreType.DMA,
        ],
    )(inp.astype(jnp.int32), table)

    return out[:, :, :EMB_DIM]


_jitted = {}


def kernel(input, word2syllable, embedding):
    dev = jax.devices()[0]
    fn = _jitted.get(dev)
    if fn is None:
        fmt = jlayout.Format(
            jlayout.Layout(major_to_minor=(0, 1, 2)),
            jax.sharding.SingleDeviceSharding(dev),
        )
        fn = jax.jit(_impl, out_shardings=fmt)
        _jitted[dev] = fn
    return fn(input, word2syllable, embedding)


# speedup vs baseline: 6.5407x; 5.1396x over previous
import functools

import jax
import jax.numpy as jnp
from jax import lax
from jax.experimental import pallas as pl
from jax.experimental.pallas import tpu as pltpu
from jax.experimental.pallas import tpu_sc as plsc

NC = 2
NS = 16
NW = NC * NS
L = 16

EMB_DIM = 64
NCLS = 50


def _cls_body(batch, hist, inp_hbm, w2s_hbm, cls_hbm, w2s_v, in_v, out_v):
    wid = lax.axis_index("s") * NC + lax.axis_index("c")
    bw = batch // NW
    b0 = pl.multiple_of(wid * bw, bw)
    pltpu.sync_copy(w2s_hbm, w2s_v)
    pltpu.sync_copy(inp_hbm.at[pl.ds(b0, bw)], in_v)
    iota = jnp.arange(L, dtype=jnp.int32)

    def j_body(j, carry):
        jv = jnp.zeros((L,), jnp.int32) + j
        for g in range(bw // L):
            widx = plsc.load_gather(in_v, [iota + g * L, jv])
            cls = plsc.load_gather(w2s_v, [widx])
            out_v[j, pl.ds(g * L, L)] = cls
        return carry

    lax.fori_loop(0, hist, j_body, 0)
    pltpu.sync_copy(out_v, cls_hbm.at[:, pl.ds(b0, bw)])


def _expand_kernel(cls_ref, emb_ref, out_ref):
    jb, batch = cls_ref.shape
    iota_c = lax.broadcasted_iota(jnp.int32, (NCLS, batch), 0)
    emb = emb_ref[...]
    for jj in range(jb):
        cls = cls_ref[jj, :].reshape(1, batch)
        onehot = jnp.where(iota_c == cls, 1.0, 0.0).astype(jnp.float32)
        out_ref[jj] = lax.dot_general(
            emb, onehot, (((0,), (0,)), ((), ())),
            preferred_element_type=jnp.float32)


def _impl(inp, w2s, emb):
    batch, hist = inp.shape
    bw = batch // NW

    mesh = plsc.VectorSubcoreMesh(core_axis_name="c", subcore_axis_name="s")
    params = pltpu.CompilerParams(needs_layout_passes=False)

    cls2 = pl.kernel(
        functools.partial(_cls_body, batch, hist),
        out_type=jax.ShapeDtypeStruct((hist, batch), jnp.int32),
        mesh=mesh,
        compiler_params=params,
        scratch_types=[
            pltpu.VMEM((w2s.shape[0],), jnp.int32),
            pltpu.VMEM((bw, hist), jnp.int32),
            pltpu.VMEM((hist, bw), jnp.int32),
        ],
    )(inp.astype(jnp.int32), w2s.astype(jnp.int32))

    JB = 8
    out_t = pl.pallas_call(
        _expand_kernel,
        grid=(hist // JB,),
        in_specs=[
            pl.BlockSpec((JB, batch), lambda j: (j, 0)),
            pl.BlockSpec((NCLS, EMB_DIM), lambda j: (0, 0)),
        ],
        out_specs=pl.BlockSpec((JB, EMB_DIM, batch), lambda j: (j, 0, 0)),
        out_shape=jax.ShapeDtypeStruct((hist, EMB_DIM, batch), jnp.float32),
    )(cls2, emb)

    return jnp.transpose(out_t, (2, 0, 1))


_jit_impl = jax.jit(_impl)


def kernel(input, word2syllable, embedding):
    return _jit_impl(input, word2syllable, embedding)
